# baseline (speedup 1.0000x reference)
import functools

import jax
import jax.numpy as jnp
from jax import lax
from jax.experimental import pallas as pl
from jax.experimental.pallas import tpu as pltpu
from jax.experimental.pallas import tpu_sc as plsc

N = 50000
D = 128
H = 64
HH = 32
E = 500000

N_PAD = 50048
ROWS_T = N_PAD // 16
P_T = 32768
E_PAD = 16 * P_T
STAGE = 1024
CHUNK = 128
N_STAGES = P_T // STAGE
N_SUB = STAGE // CHUNK

_mesh = plsc.VectorSubcoreMesh(core_axis_name="c", subcore_axis_name="s")
_sc_params = pltpu.CompilerParams(use_tc_tiling_on_sc=False)


ROWS_PER_TILE = P_T // CHUNK
NBUF = 4
WIN = 16
NW_T = ROWS_PER_TILE // WIN


def _segsum_pipeline(c, s, h_ref, ids_ref, acc, iw0, iw1, rows, semg,
                     semi0, semi1):
    roff = s * ROWS_PER_TILE
    pltpu.sync_copy(ids_ref.at[pl.ds(roff, WIN)], iw0)
    pltpu.async_copy(ids_ref.at[pl.ds(roff + WIN, WIN)], iw1, semi1)

    for b in range(NBUF):
        pltpu.async_copy(h_ref.at[c].at[iw0.at[b, 0]], rows[b], semg[b])

    def process(w, iw_cur, iw_nxt, semi_cur, semi_nxt):
        for k in range(WIN):
            b = k % NBUF
            pltpu.make_async_copy(
                h_ref.at[c].at[iw_cur.at[k, 0]], rows[b], semg[b]).wait()
            pltpu.sync_copy(rows[b], acc.at[iw_cur.at[k, 1]], add=True)
            if k == WIN - NBUF:
                @pl.when(w < NW_T - 1)
                def _wait_idx():
                    pltpu.make_async_copy(
                        ids_ref.at[pl.ds(roff, WIN)], iw_nxt, semi_nxt).wait()
            if k < WIN - NBUF:
                pltpu.async_copy(
                    h_ref.at[c].at[iw_cur.at[k + NBUF, 0]], rows[b], semg[b])
            else:
                @pl.when(w < NW_T - 1)
                def _fire_nxt():
                    pltpu.async_copy(
                        h_ref.at[c].at[iw_nxt.at[k + NBUF - WIN, 0]],
                        rows[b], semg[b])
        @pl.when(w < NW_T - 2)
        def _prefetch_idx():
            pltpu.async_copy(
                ids_ref.at[pl.ds(roff + (w + 2) * WIN, WIN)], iw_cur, semi_cur)

    def dbl(g, _):
        process(2 * g, iw0, iw1, semi0, semi1)
        process(2 * g + 1, iw1, iw0, semi1, semi0)
        return 0

    lax.fori_loop(0, NW_T // 2, dbl, 0)


def _segsum_body(hu_ref, hi_ref, ids_ui_ref, ids_iu_ref, zer_ref,
                 mi_ref, mu_ref,
                 acc0, acc1, iw0, iw1,
                 rows0, rows1, rows2, rows3,
                 semg0, semg1, semg2, semg3,
                 semi0, semi1):
    c = lax.axis_index("c")
    s = lax.axis_index("s")
    rows = (rows0, rows1, rows2, rows3)
    semg = (semg0, semg1, semg2, semg3)
    pltpu.sync_copy(zer_ref, acc0.at[pl.ds(s * ROWS_T, ROWS_T)])
    pltpu.sync_copy(zer_ref, acc1.at[pl.ds(s * ROWS_T, ROWS_T)])
    plsc.subcore_barrier()
    _segsum_pipeline(c, s, hu_ref, ids_ui_ref, acc0, iw0, iw1, rows, semg,
                     semi0, semi1)
    _segsum_pipeline(c, s, hi_ref, ids_iu_ref, acc1, iw0, iw1, rows, semg,
                     semi0, semi1)
    plsc.subcore_barrier()
    pltpu.sync_copy(acc0.at[pl.ds(s * ROWS_T, ROWS_T)],
                    mi_ref.at[c].at[pl.ds(s * ROWS_T, ROWS_T)])
    pltpu.sync_copy(acc1.at[pl.ds(s * ROWS_T, ROWS_T)],
                    mu_ref.at[c].at[pl.ds(s * ROWS_T, ROWS_T)])


_segsum = functools.partial(
    pl.kernel,
    _segsum_body,
    out_type=[jax.ShapeDtypeStruct((2, N_PAD, HH), jnp.bfloat16),
              jax.ShapeDtypeStruct((2, N_PAD, HH), jnp.bfloat16)],
    mesh=_mesh,
    scratch_types=(
        [pltpu.VMEM_SHARED((N_PAD, HH), jnp.bfloat16)] * 2
        + [pltpu.VMEM((WIN, 2, CHUNK), jnp.int32)] * 2
        + [pltpu.VMEM((CHUNK, HH), jnp.bfloat16)] * NBUF
        + [pltpu.SemaphoreType.DMA] * (NBUF + 2)
    ),
    compiler_params=_sc_params,
)()


def _count_body(dsts_ref, ones_ref, zer_ref, out_ref,
                acc, dst_st, ones_v):
    c = lax.axis_index("c")
    s = lax.axis_index("s")
    pltpu.sync_copy(ones_ref, ones_v)
    pltpu.sync_copy(zer_ref, acc.at[pl.ds(s * ROWS_T, ROWS_T)])
    plsc.subcore_barrier()
    rows_per_tile = P_T // CHUNK

    def stage(st, _):
        roff = s * rows_per_tile + st * N_SUB
        pltpu.sync_copy(dsts_ref.at[c].at[pl.ds(roff, N_SUB)], dst_st)

        def sub(j, _):
            pltpu.sync_copy(ones_v, acc.at[dst_st.at[j]], add=True)
            return 0

        lax.fori_loop(0, N_SUB, sub, 0)
        return 0

    lax.fori_loop(0, N_STAGES, stage, 0)
    plsc.subcore_barrier()
    pltpu.sync_copy(acc.at[pl.ds(s * ROWS_T, ROWS_T)],
                    out_ref.at[c].at[pl.ds(s * ROWS_T, ROWS_T)])


_count = functools.partial(
    pl.kernel,
    _count_body,
    out_type=jax.ShapeDtypeStruct((2, N_PAD, 16), jnp.float32),
    mesh=_mesh,
    scratch_types=[
        pltpu.VMEM_SHARED((N_PAD, 16), jnp.float32),
        pltpu.VMEM((N_SUB, CHUNK), jnp.int32),
        pltpu.VMEM((CHUNK, 16), jnp.float32),
    ],
    compiler_params=_sc_params,
)()


def _enc_body(x_ref, w_ref, b_ref, out_ref):
    h = jnp.maximum(
        jnp.dot(x_ref[...], w_ref[...], preferred_element_type=jnp.float32)
        + b_ref[...], 0.0)
    out_ref[...] = jnp.stack([h[:, :HH], h[:, HH:]])


def _encode(x, w, b):
    return pl.pallas_call(
        _enc_body,
        grid=(16,),
        in_specs=[
            pl.BlockSpec((ROWS_T, D), lambda i: (i, 0)),
            pl.BlockSpec((D, H), lambda i: (0, 0)),
            pl.BlockSpec((1, H), lambda i: (0, 0)),
        ],
        out_specs=pl.BlockSpec((2, ROWS_T, HH), lambda i: (0, i, 0)),
        out_shape=jax.ShapeDtypeStruct((2, N_PAD, HH), jnp.float32),
    )(x, w, b.reshape(1, H))


def _layer_body(m_ref, c_ref, h_ref, wn_ref, bn_ref, wr_ref,
                out_s_ref, out_f_ref):
    m = jnp.concatenate([m_ref[0], m_ref[1]], axis=1)
    h = jnp.concatenate([h_ref[0], h_ref[1]], axis=1)
    inv = 1.0 / jnp.maximum(c_ref[0][:, 0:1], 1.0)
    o = (jnp.dot(m * inv, wn_ref[...], preferred_element_type=jnp.float32)
         + jnp.dot(h, wr_ref[...], preferred_element_type=jnp.float32)
         + bn_ref[...])
    o = jnp.maximum(o, 0.0)
    out_s_ref[...] = jnp.stack([o[:, :HH], o[:, HH:]])
    out_f_ref[...] = o


def _layer(m_split, cnt, plane, h_split, wn, bn, wr):
    return pl.pallas_call(
        _layer_body,
        grid=(16,),
        in_specs=[
            pl.BlockSpec((2, ROWS_T, HH), lambda i: (0, i, 0)),
            pl.BlockSpec((1, ROWS_T, 16), lambda i, p=plane: (p, i, 0)),
            pl.BlockSpec((2, ROWS_T, HH), lambda i: (0, i, 0)),
            pl.BlockSpec((H, H), lambda i: (0, 0)),
            pl.BlockSpec((1, H), lambda i: (0, 0)),
            pl.BlockSpec((H, H), lambda i: (0, 0)),
        ],
        out_specs=[
            pl.BlockSpec((2, ROWS_T, HH), lambda i: (0, i, 0)),
            pl.BlockSpec((ROWS_T, H), lambda i: (i, 0)),
        ],
        out_shape=[
            jax.ShapeDtypeStruct((2, N_PAD, HH), jnp.float32),
            jax.ShapeDtypeStruct((N_PAD, H), jnp.float32),
        ],
    )(m_split, cnt, h_split, wn, bn.reshape(1, H), wr)


def _pad_edges(idx):
    src = jnp.concatenate(
        [idx[0], jnp.zeros((E_PAD - E,), jnp.int32)]).reshape(-1, 1, CHUNK)
    dst = jnp.concatenate(
        [idx[1], jnp.full((E_PAD - E,), N, jnp.int32)]).reshape(-1, 1, CHUNK)
    return jnp.concatenate([src, dst], axis=1)


def kernel(x_user, x_item, edge_index_ui, edge_index_iu,
           enc_W_user, enc_b_user, enc_W_item, enc_b_item,
           l1_ui_Wn, l1_ui_bn, l1_ui_Wr, l1_iu_Wn, l1_iu_bn, l1_iu_Wr,
           l2_ui_Wn, l2_ui_bn, l2_ui_Wr, l2_iu_Wn, l2_iu_bn, l2_iu_Wr):
    pad = ((0, N_PAD - N), (0, 0))
    xu = jnp.pad(x_user, pad)
    xi = jnp.pad(x_item, pad)

    ids_ui = _pad_edges(edge_index_ui)
    ids_iu = _pad_edges(edge_index_iu)
    dsts = jnp.stack([ids_ui[:, 1], ids_iu[:, 1]])

    zer32 = jnp.zeros((ROWS_T, HH), jnp.bfloat16)
    zer16 = jnp.zeros((ROWS_T, 16), jnp.float32)
    ones16 = jnp.ones((CHUNK, 16), jnp.float32)

    hu = _encode(xu, enc_W_user, enc_b_user)
    hi = _encode(xi, enc_W_item, enc_b_item)
    cnt = _count(dsts, ones16, zer16)

    for (ui_Wn, ui_bn, ui_Wr, iu_Wn, iu_bn, iu_Wr) in (
            (l1_ui_Wn, l1_ui_bn, l1_ui_Wr, l1_iu_Wn, l1_iu_bn, l1_iu_Wr),
            (l2_ui_Wn, l2_ui_bn, l2_ui_Wr, l2_iu_Wn, l2_iu_bn, l2_iu_Wr)):
        m_item, m_user = _segsum(hu.astype(jnp.bfloat16),
                                 hi.astype(jnp.bfloat16),
                                 ids_ui, ids_iu, zer32)
        hi, hi_full = _layer(m_item.astype(jnp.float32), cnt, 0,
                             hi, ui_Wn, ui_bn, ui_Wr)
        hu, hu_full = _layer(m_user.astype(jnp.float32), cnt, 1,
                             hu, iu_Wn, iu_bn, iu_Wr)

    return (hu_full[:N], hi_full[:N])


# speedup vs baseline: 1.1238x; 1.1238x over previous
import functools

import jax
import jax.numpy as jnp
from jax import lax
from jax.experimental import pallas as pl
from jax.experimental.pallas import tpu as pltpu
from jax.experimental.pallas import tpu_sc as plsc

N = 50000
D = 128
H = 64
HH = 32
E = 500000

N_PAD = 50048
ROWS_T = N_PAD // 16
P_T = 32768
E_PAD = 16 * P_T
STAGE = 1024
CHUNK = 128
N_STAGES = P_T // STAGE
N_SUB = STAGE // CHUNK

_mesh = plsc.VectorSubcoreMesh(core_axis_name="c", subcore_axis_name="s")
_sc_params = pltpu.CompilerParams(use_tc_tiling_on_sc=False)


ROWS_PER_TILE = P_T // CHUNK
NBUF = 8
WIN = 16
NW_T = ROWS_PER_TILE // WIN


def _segsum_body(h_ref, ids_ref, zer_ref, out_ref,
                 acc, iw0, iw1,
                 rows0, rows1, rows2, rows3, rows4, rows5, rows6, rows7,
                 semg0, semg1, semg2, semg3, semg4, semg5, semg6, semg7,
                 semi0, semi1):
    c = lax.axis_index("c")
    s = lax.axis_index("s")
    rows = (rows0, rows1, rows2, rows3, rows4, rows5, rows6, rows7)
    semg = (semg0, semg1, semg2, semg3, semg4, semg5, semg6, semg7)
    pltpu.sync_copy(zer_ref, acc.at[pl.ds(s * ROWS_T, ROWS_T)])
    roff = s * ROWS_PER_TILE
    pltpu.sync_copy(ids_ref.at[pl.ds(roff, WIN)], iw0)
    pltpu.async_copy(ids_ref.at[pl.ds(roff + WIN, WIN)], iw1, semi1)
    plsc.subcore_barrier()

    for b in range(NBUF):
        pltpu.async_copy(h_ref.at[c].at[iw0.at[b, 0]], rows[b], semg[b])

    def process(w, iw_cur, iw_nxt, semi_cur, semi_nxt):
        for k in range(WIN):
            b = k % NBUF
            pltpu.make_async_copy(
                h_ref.at[c].at[iw_cur.at[k, 0]], rows[b], semg[b]).wait()
            pltpu.sync_copy(rows[b], acc.at[iw_cur.at[k, 1]], add=True)
            if k == WIN - NBUF:
                @pl.when(w < NW_T - 1)
                def _wait_idx():
                    pltpu.make_async_copy(
                        ids_ref.at[pl.ds(roff, WIN)], iw_nxt, semi_nxt).wait()
            if k < WIN - NBUF:
                pltpu.async_copy(
                    h_ref.at[c].at[iw_cur.at[k + NBUF, 0]], rows[b], semg[b])
            else:
                @pl.when(w < NW_T - 1)
                def _fire_nxt():
                    pltpu.async_copy(
                        h_ref.at[c].at[iw_nxt.at[k + NBUF - WIN, 0]],
                        rows[b], semg[b])
        @pl.when(w < NW_T - 2)
        def _prefetch_idx():
            pltpu.async_copy(
                ids_ref.at[pl.ds(roff + (w + 2) * WIN, WIN)], iw_cur, semi_cur)

    def dbl(g, _):
        process(2 * g, iw0, iw1, semi0, semi1)
        process(2 * g + 1, iw1, iw0, semi1, semi0)
        return 0

    lax.fori_loop(0, NW_T // 2, dbl, 0)
    plsc.subcore_barrier()
    pltpu.sync_copy(acc.at[pl.ds(s * ROWS_T, ROWS_T)],
                    out_ref.at[c].at[pl.ds(s * ROWS_T, ROWS_T)])


_segsum = functools.partial(
    pl.kernel,
    _segsum_body,
    out_type=jax.ShapeDtypeStruct((2, N_PAD, HH), jnp.bfloat16),
    mesh=_mesh,
    scratch_types=[
        pltpu.VMEM_SHARED((N_PAD, HH), jnp.bfloat16),
        pltpu.VMEM((WIN, 2, CHUNK), jnp.int32),
        pltpu.VMEM((WIN, 2, CHUNK), jnp.int32),
    ] + [pltpu.VMEM((CHUNK, HH), jnp.bfloat16)] * 8
      + [pltpu.SemaphoreType.DMA] * 10,
    compiler_params=_sc_params,
)()


def _count_body(dsts_ref, ones_ref, zer_ref, out_ref,
                acc, dst_st, ones_v):
    c = lax.axis_index("c")
    s = lax.axis_index("s")
    pltpu.sync_copy(ones_ref, ones_v)
    pltpu.sync_copy(zer_ref, acc.at[pl.ds(s * ROWS_T, ROWS_T)])
    plsc.subcore_barrier()
    rows_per_tile = P_T // CHUNK

    def stage(st, _):
        roff = s * rows_per_tile + st * N_SUB
        pltpu.sync_copy(dsts_ref.at[c].at[pl.ds(roff, N_SUB)], dst_st)

        def sub(j, _):
            pltpu.sync_copy(ones_v, acc.at[dst_st.at[j]], add=True)
            return 0

        lax.fori_loop(0, N_SUB, sub, 0)
        return 0

    lax.fori_loop(0, N_STAGES, stage, 0)
    plsc.subcore_barrier()
    pltpu.sync_copy(acc.at[pl.ds(s * ROWS_T, ROWS_T)],
                    out_ref.at[c].at[pl.ds(s * ROWS_T, ROWS_T)])


_count = functools.partial(
    pl.kernel,
    _count_body,
    out_type=jax.ShapeDtypeStruct((2, N_PAD, 16), jnp.float32),
    mesh=_mesh,
    scratch_types=[
        pltpu.VMEM_SHARED((N_PAD, 16), jnp.float32),
        pltpu.VMEM((N_SUB, CHUNK), jnp.int32),
        pltpu.VMEM((CHUNK, 16), jnp.float32),
    ],
    compiler_params=_sc_params,
)()


def _enc_body(x_ref, w_ref, b_ref, out_ref):
    h = jnp.maximum(
        jnp.dot(x_ref[...], w_ref[...], preferred_element_type=jnp.float32)
        + b_ref[...], 0.0)
    out_ref[...] = jnp.stack([h[:, :HH], h[:, HH:]])


def _encode(x, w, b):
    return pl.pallas_call(
        _enc_body,
        grid=(16,),
        in_specs=[
            pl.BlockSpec((ROWS_T, D), lambda i: (i, 0)),
            pl.BlockSpec((D, H), lambda i: (0, 0)),
            pl.BlockSpec((1, H), lambda i: (0, 0)),
        ],
        out_specs=pl.BlockSpec((2, ROWS_T, HH), lambda i: (0, i, 0)),
        out_shape=jax.ShapeDtypeStruct((2, N_PAD, HH), jnp.float32),
    )(x, w, b.reshape(1, H))


def _layer_body(m_ref, c_ref, h_ref, wn_ref, bn_ref, wr_ref,
                out_s_ref, out_f_ref):
    m = jnp.concatenate([m_ref[0], m_ref[1]], axis=1)
    h = jnp.concatenate([h_ref[0], h_ref[1]], axis=1)
    inv = 1.0 / jnp.maximum(c_ref[0][:, 0:1], 1.0)
    o = (jnp.dot(m * inv, wn_ref[...], preferred_element_type=jnp.float32)
         + jnp.dot(h, wr_ref[...], preferred_element_type=jnp.float32)
         + bn_ref[...])
    o = jnp.maximum(o, 0.0)
    out_s_ref[...] = jnp.stack([o[:, :HH], o[:, HH:]])
    out_f_ref[...] = o


def _layer(m_split, cnt, plane, h_split, wn, bn, wr):
    return pl.pallas_call(
        _layer_body,
        grid=(16,),
        in_specs=[
            pl.BlockSpec((2, ROWS_T, HH), lambda i: (0, i, 0)),
            pl.BlockSpec((1, ROWS_T, 16), lambda i, p=plane: (p, i, 0)),
            pl.BlockSpec((2, ROWS_T, HH), lambda i: (0, i, 0)),
            pl.BlockSpec((H, H), lambda i: (0, 0)),
            pl.BlockSpec((1, H), lambda i: (0, 0)),
            pl.BlockSpec((H, H), lambda i: (0, 0)),
        ],
        out_specs=[
            pl.BlockSpec((2, ROWS_T, HH), lambda i: (0, i, 0)),
            pl.BlockSpec((ROWS_T, H), lambda i: (i, 0)),
        ],
        out_shape=[
            jax.ShapeDtypeStruct((2, N_PAD, HH), jnp.float32),
            jax.ShapeDtypeStruct((N_PAD, H), jnp.float32),
        ],
    )(m_split, cnt, h_split, wn, bn.reshape(1, H), wr)


def _pad_edges(idx):
    src = jnp.concatenate(
        [idx[0], jnp.zeros((E_PAD - E,), jnp.int32)]).reshape(-1, 1, CHUNK)
    dst = jnp.concatenate(
        [idx[1], jnp.full((E_PAD - E,), N, jnp.int32)]).reshape(-1, 1, CHUNK)
    return jnp.concatenate([src, dst], axis=1)


def kernel(x_user, x_item, edge_index_ui, edge_index_iu,
           enc_W_user, enc_b_user, enc_W_item, enc_b_item,
           l1_ui_Wn, l1_ui_bn, l1_ui_Wr, l1_iu_Wn, l1_iu_bn, l1_iu_Wr,
           l2_ui_Wn, l2_ui_bn, l2_ui_Wr, l2_iu_Wn, l2_iu_bn, l2_iu_Wr):
    pad = ((0, N_PAD - N), (0, 0))
    xu = jnp.pad(x_user, pad)
    xi = jnp.pad(x_item, pad)

    ids_ui = _pad_edges(edge_index_ui)
    ids_iu = _pad_edges(edge_index_iu)
    dsts = jnp.stack([ids_ui[:, 1], ids_iu[:, 1]])

    zer32 = jnp.zeros((ROWS_T, HH), jnp.bfloat16)
    zer16 = jnp.zeros((ROWS_T, 16), jnp.float32)
    ones16 = jnp.ones((CHUNK, 16), jnp.float32)

    hu = _encode(xu, enc_W_user, enc_b_user)
    hi = _encode(xi, enc_W_item, enc_b_item)
    cnt = _count(dsts, ones16, zer16)

    for (ui_Wn, ui_bn, ui_Wr, iu_Wn, iu_bn, iu_Wr) in (
            (l1_ui_Wn, l1_ui_bn, l1_ui_Wr, l1_iu_Wn, l1_iu_bn, l1_iu_Wr),
            (l2_ui_Wn, l2_ui_bn, l2_ui_Wr, l2_iu_Wn, l2_iu_bn, l2_iu_Wr)):
        m_item = _segsum(hu.astype(jnp.bfloat16), ids_ui, zer32)
        m_user = _segsum(hi.astype(jnp.bfloat16), ids_iu, zer32)
        hi, hi_full = _layer(m_item.astype(jnp.float32), cnt, 0,
                             hi, ui_Wn, ui_bn, ui_Wr)
        hu, hu_full = _layer(m_user.astype(jnp.float32), cnt, 1,
                             hu, iu_Wn, iu_bn, iu_Wr)

    return (hu_full[:N], hi_full[:N])


# speedup vs baseline: 1.3862x; 1.2335x over previous
import functools

import jax
import jax.numpy as jnp
from jax import lax
from jax.experimental import pallas as pl
from jax.experimental.pallas import tpu as pltpu
from jax.experimental.pallas import tpu_sc as plsc

N = 50000
D = 128
H = 64
HH = 32
E = 500000

N_PAD = 51200
ROWS_T = N_PAD // 16
P_T = 32768
E_PAD = 16 * P_T
STAGE = 1024
CHUNK = 128
N_STAGES = P_T // STAGE
N_SUB = STAGE // CHUNK

_mesh = plsc.VectorSubcoreMesh(core_axis_name="c", subcore_axis_name="s")
_sc_params = pltpu.CompilerParams(use_tc_tiling_on_sc=False)


ROWS_PER_TILE = P_T // CHUNK
NBUF = 8
WIN = 16
NW_T = ROWS_PER_TILE // WIN


def _segsum_body(h_ref, ids_ref, zer_ref, out_ref,
                 acc, iw0, iw1,
                 rows0, rows1, rows2, rows3, rows4, rows5, rows6, rows7,
                 semg0, semg1, semg2, semg3, semg4, semg5, semg6, semg7,
                 semi0, semi1):
    c = lax.axis_index("c")
    s = lax.axis_index("s")
    rows = (rows0, rows1, rows2, rows3, rows4, rows5, rows6, rows7)
    semg = (semg0, semg1, semg2, semg3, semg4, semg5, semg6, semg7)
    pltpu.sync_copy(zer_ref, acc.at[pl.ds(s * ROWS_T, ROWS_T)])
    roff = s * ROWS_PER_TILE
    pltpu.sync_copy(ids_ref.at[pl.ds(roff, WIN)], iw0)
    pltpu.async_copy(ids_ref.at[pl.ds(roff + WIN, WIN)], iw1, semi1)
    plsc.subcore_barrier()

    for b in range(NBUF):
        pltpu.async_copy(h_ref.at[c].at[iw0.at[b, 0]], rows[b], semg[b])

    def process(w, iw_cur, iw_nxt, semi_cur, semi_nxt):
        for k in range(WIN):
            b = k % NBUF
            pltpu.make_async_copy(
                h_ref.at[c].at[iw_cur.at[k, 0]], rows[b], semg[b]).wait()
            pltpu.sync_copy(rows[b], acc.at[iw_cur.at[k, 1]], add=True)
            if k == WIN - NBUF:
                @pl.when(w < NW_T - 1)
                def _wait_idx():
                    pltpu.make_async_copy(
                        ids_ref.at[pl.ds(roff, WIN)], iw_nxt, semi_nxt).wait()
            if k < WIN - NBUF:
                pltpu.async_copy(
                    h_ref.at[c].at[iw_cur.at[k + NBUF, 0]], rows[b], semg[b])
            else:
                @pl.when(w < NW_T - 1)
                def _fire_nxt():
                    pltpu.async_copy(
                        h_ref.at[c].at[iw_nxt.at[k + NBUF - WIN, 0]],
                        rows[b], semg[b])
        @pl.when(w < NW_T - 2)
        def _prefetch_idx():
            pltpu.async_copy(
                ids_ref.at[pl.ds(roff + (w + 2) * WIN, WIN)], iw_cur, semi_cur)

    def dbl(g, _):
        process(2 * g, iw0, iw1, semi0, semi1)
        process(2 * g + 1, iw1, iw0, semi1, semi0)
        return 0

    lax.fori_loop(0, NW_T // 2, dbl, 0)
    plsc.subcore_barrier()
    pltpu.sync_copy(acc.at[pl.ds(s * ROWS_T, ROWS_T)],
                    out_ref.at[c].at[pl.ds(s * ROWS_T, ROWS_T)])


_segsum = functools.partial(
    pl.kernel,
    _segsum_body,
    out_type=jax.ShapeDtypeStruct((2, N_PAD, HH), jnp.bfloat16),
    mesh=_mesh,
    scratch_types=[
        pltpu.VMEM_SHARED((N_PAD, HH), jnp.bfloat16),
        pltpu.VMEM((WIN, 2, CHUNK), jnp.int32),
        pltpu.VMEM((WIN, 2, CHUNK), jnp.int32),
    ] + [pltpu.VMEM((CHUNK, HH), jnp.bfloat16)] * 8
      + [pltpu.SemaphoreType.DMA] * 10,
    compiler_params=_sc_params,
)()


def _count_body(dsts_ref, ones_ref, zer_ref, out_ref,
                acc, dst_st, ones_v):
    c = lax.axis_index("c")
    s = lax.axis_index("s")
    pltpu.sync_copy(ones_ref, ones_v)
    pltpu.sync_copy(zer_ref, acc.at[pl.ds(s * ROWS_T, ROWS_T)])
    plsc.subcore_barrier()
    rows_per_tile = P_T // CHUNK

    def stage(st, _):
        roff = s * rows_per_tile + st * N_SUB
        pltpu.sync_copy(dsts_ref.at[c].at[pl.ds(roff, N_SUB)], dst_st)

        def sub(j, _):
            pltpu.sync_copy(ones_v, acc.at[dst_st.at[j]], add=True)
            return 0

        lax.fori_loop(0, N_SUB, sub, 0)
        return 0

    lax.fori_loop(0, N_STAGES, stage, 0)
    plsc.subcore_barrier()
    pltpu.sync_copy(acc.at[pl.ds(s * ROWS_T, ROWS_T)],
                    out_ref.at[c].at[pl.ds(s * ROWS_T, ROWS_T)])


_count = functools.partial(
    pl.kernel,
    _count_body,
    out_type=jax.ShapeDtypeStruct((2, N_PAD, 16), jnp.float32),
    mesh=_mesh,
    scratch_types=[
        pltpu.VMEM_SHARED((N_PAD, 16), jnp.float32),
        pltpu.VMEM((N_SUB, CHUNK), jnp.int32),
        pltpu.VMEM((CHUNK, 16), jnp.float32),
    ],
    compiler_params=_sc_params,
)()


def _enc_body(x_ref, w_ref, b_ref, out_ref):
    h = jnp.maximum(
        jnp.dot(x_ref[...], w_ref[...], preferred_element_type=jnp.float32)
        + b_ref[...], 0.0)
    out_ref[...] = jnp.stack([h[:, :HH], h[:, HH:]]).astype(jnp.bfloat16)


def _encode(x, w, b):
    return pl.pallas_call(
        _enc_body,
        grid=(16,),
        in_specs=[
            pl.BlockSpec((ROWS_T, D), lambda i: (i, 0)),
            pl.BlockSpec((D, H), lambda i: (0, 0)),
            pl.BlockSpec((1, H), lambda i: (0, 0)),
        ],
        out_specs=pl.BlockSpec((2, ROWS_T, HH), lambda i: (0, i, 0)),
        out_shape=jax.ShapeDtypeStruct((2, N_PAD, HH), jnp.bfloat16),
    )(x, w, b.reshape(1, H))


def _layer_body(m_ref, c_ref, h_ref, wn_ref, bn_ref, wr_ref,
                out_s_ref, out_f_ref):
    m = jnp.concatenate([m_ref[0], m_ref[1]], axis=1).astype(jnp.float32)
    h = jnp.concatenate([h_ref[0], h_ref[1]], axis=1).astype(jnp.float32)
    inv = 1.0 / jnp.maximum(c_ref[0][:, 0:1], 1.0)
    o = (jnp.dot(m * inv, wn_ref[...], preferred_element_type=jnp.float32)
         + jnp.dot(h, wr_ref[...], preferred_element_type=jnp.float32)
         + bn_ref[...])
    o = jnp.maximum(o, 0.0)
    out_s_ref[...] = jnp.stack([o[:, :HH], o[:, HH:]]).astype(jnp.bfloat16)
    out_f_ref[...] = o


def _layer(m_split, cnt, plane, h_split, wn, bn, wr):
    return pl.pallas_call(
        _layer_body,
        grid=(16,),
        in_specs=[
            pl.BlockSpec((2, ROWS_T, HH), lambda i: (0, i, 0)),
            pl.BlockSpec((1, ROWS_T, 16), lambda i, p=plane: (p, i, 0)),
            pl.BlockSpec((2, ROWS_T, HH), lambda i: (0, i, 0)),
            pl.BlockSpec((H, H), lambda i: (0, 0)),
            pl.BlockSpec((1, H), lambda i: (0, 0)),
            pl.BlockSpec((H, H), lambda i: (0, 0)),
        ],
        out_specs=[
            pl.BlockSpec((2, ROWS_T, HH), lambda i: (0, i, 0)),
            pl.BlockSpec((ROWS_T, H), lambda i: (i, 0)),
        ],
        out_shape=[
            jax.ShapeDtypeStruct((2, N_PAD, HH), jnp.bfloat16),
            jax.ShapeDtypeStruct((N_PAD, H), jnp.float32),
        ],
    )(m_split, cnt, h_split, wn, bn.reshape(1, H), wr)


def _pad_edges(idx):
    src = jnp.concatenate(
        [idx[0], jnp.zeros((E_PAD - E,), jnp.int32)]).reshape(-1, 1, CHUNK)
    dst = jnp.concatenate(
        [idx[1], jnp.full((E_PAD - E,), N, jnp.int32)]).reshape(-1, 1, CHUNK)
    return jnp.concatenate([src, dst], axis=1)


def kernel(x_user, x_item, edge_index_ui, edge_index_iu,
           enc_W_user, enc_b_user, enc_W_item, enc_b_item,
           l1_ui_Wn, l1_ui_bn, l1_ui_Wr, l1_iu_Wn, l1_iu_bn, l1_iu_Wr,
           l2_ui_Wn, l2_ui_bn, l2_ui_Wr, l2_iu_Wn, l2_iu_bn, l2_iu_Wr):
    pad = ((0, N_PAD - N), (0, 0))
    xu = jnp.pad(x_user, pad)
    xi = jnp.pad(x_item, pad)

    ids_ui = _pad_edges(edge_index_ui)
    ids_iu = _pad_edges(edge_index_iu)
    dsts = jnp.stack([ids_ui[:, 1], ids_iu[:, 1]])

    zer32 = jnp.zeros((ROWS_T, HH), jnp.bfloat16)
    zer16 = jnp.zeros((ROWS_T, 16), jnp.float32)
    ones16 = jnp.ones((CHUNK, 16), jnp.float32)

    hu = _encode(xu, enc_W_user, enc_b_user)
    hi = _encode(xi, enc_W_item, enc_b_item)
    cnt = _count(dsts, ones16, zer16)

    for (ui_Wn, ui_bn, ui_Wr, iu_Wn, iu_bn, iu_Wr) in (
            (l1_ui_Wn, l1_ui_bn, l1_ui_Wr, l1_iu_Wn, l1_iu_bn, l1_iu_Wr),
            (l2_ui_Wn, l2_ui_bn, l2_ui_Wr, l2_iu_Wn, l2_iu_bn, l2_iu_Wr)):
        m_item = _segsum(hu, ids_ui, zer32)
        m_user = _segsum(hi, ids_iu, zer32)
        hi, hi_full = _layer(m_item, cnt, 0, hi, ui_Wn, ui_bn, ui_Wr)
        hu, hu_full = _layer(m_user, cnt, 1, hu, iu_Wn, iu_bn, iu_Wr)

    return (hu_full[:N], hi_full[:N])
